# baseline (speedup 1.0000x reference)
import jax
import jax.numpy as jnp
from jax import lax
from jax.experimental import pallas as pl
from jax.experimental.pallas import tpu as pltpu
from jax.experimental.pallas import tpu_sc as plsc

D = 32
NC, NS = 2, 16
NW = NC * NS
CHUNK = 256
K = 5
B = 16384 * 50
BPW = B // NW
NSTEPS = BPW // CHUNK
NROUNDS = NSTEPS // K


def _emb_body(idx_hbm, table_hbm, out_hbm, idx_v, *scr):
    rows = [[scr[g * K + t] for t in range(K)] for g in range(2)]
    gsem = [scr[2 * K], scr[2 * K + 1]]
    wsem = [scr[2 * K + 2], scr[2 * K + 3]]

    wid = lax.axis_index("s") * NC + lax.axis_index("c")
    pltpu.sync_copy(idx_hbm.at[wid], idx_v)

    def g_copy(g, t, r):
        step = r * K + t
        return pltpu.make_async_copy(
            table_hbm.at[idx_v.at[step]], rows[g][t], gsem[g])

    def w_copy(g, t, r):
        step = r * K + t
        return pltpu.make_async_copy(
            rows[g][t], out_hbm.at[wid, step], wsem[g])

    def fire_g(g, r):
        for t in range(K):
            g_copy(g, t, r).start()

    def drain_g(g, r):
        for t in range(K):
            g_copy(g, t, r).wait()

    def fire_w(g, r):
        for t in range(K):
            w_copy(g, t, r).start()

    def drain_w(g, r):
        for t in range(K):
            w_copy(g, t, r).wait()

    fire_g(0, 0)

    def process(g, r):
        @pl.when(r >= 1)
        def _():
            drain_w(1 - g, r - 1)

        @pl.when(r + 1 < NROUNDS)
        def _():
            fire_g(1 - g, r + 1)

        drain_g(g, r)
        fire_w(g, r)

    def body(i, carry):
        process(0, 2 * i)
        process(1, 2 * i + 1)
        return carry

    lax.fori_loop(0, NROUNDS // 2, body, 0)
    drain_w((NROUNDS - 1) % 2, NROUNDS - 1)


def kernel(token_ids, embedding_matrix):
    idx = token_ids.reshape(NW, NSTEPS, CHUNK).astype(jnp.int32)
    call = pl.kernel(
        _emb_body,
        out_type=jax.ShapeDtypeStruct((NW, NSTEPS, CHUNK, D), jnp.float32),
        mesh=plsc.VectorSubcoreMesh(core_axis_name="c", subcore_axis_name="s"),
        compiler_params=pltpu.CompilerParams(use_tc_tiling_on_sc=False),
        scratch_types=(
            [pltpu.VMEM((NSTEPS, CHUNK), jnp.int32)]
            + [pltpu.VMEM((CHUNK, D), jnp.float32) for _ in range(2 * K)]
            + [pltpu.SemaphoreType.DMA] * 4
        ),
    )
    out = call(idx, embedding_matrix)
    return out.reshape(token_ids.shape + (D,))


# speedup vs baseline: 1.3875x; 1.3875x over previous
import jax
import jax.numpy as jnp
from jax import lax
from jax.experimental import pallas as pl
from jax.experimental.pallas import tpu as pltpu
from jax.experimental.pallas import tpu_sc as plsc

D = 32
T = 50
R = 16384
NC, NS = 2, 16
NW = NC * NS
RPW = R // NW
K = 8
NROUNDS = RPW // K


def _emb_body(idx_hbm, table_hbm, out_hbm, idx_v, *scr):
    rows = [[scr[g * K + t] for t in range(K)] for g in range(2)]
    gsem = [scr[2 * K], scr[2 * K + 1]]
    wsem = [scr[2 * K + 2], scr[2 * K + 3]]

    wid = lax.axis_index("s") * NC + lax.axis_index("c")
    base = wid * RPW
    pltpu.sync_copy(idx_hbm.at[pl.ds(base, RPW)], idx_v)

    def g_copy(g, t, r):
        step = r * K + t
        return pltpu.make_async_copy(
            table_hbm.at[idx_v.at[step]], rows[g][t], gsem[g])

    def w_copy(g, t, r):
        step = r * K + t
        return pltpu.make_async_copy(
            rows[g][t], out_hbm.at[base + step], wsem[g])

    def fire_g(g, r):
        for t in range(K):
            g_copy(g, t, r).start()

    def drain_g(g, r):
        for t in range(K):
            g_copy(g, t, r).wait()

    def fire_w(g, r):
        for t in range(K):
            w_copy(g, t, r).start()

    def drain_w(g, r):
        for t in range(K):
            w_copy(g, t, r).wait()

    fire_g(0, 0)

    def process(g, r):
        @pl.when(r >= 1)
        def _():
            drain_w(1 - g, r - 1)

        @pl.when(r + 1 < NROUNDS)
        def _():
            fire_g(1 - g, r + 1)

        drain_g(g, r)
        fire_w(g, r)

    def body(i, carry):
        process(0, 2 * i)
        process(1, 2 * i + 1)
        return carry

    lax.fori_loop(0, NROUNDS // 2, body, 0)
    drain_w((NROUNDS - 1) % 2, NROUNDS - 1)


def kernel(token_ids, embedding_matrix):
    call = pl.kernel(
        _emb_body,
        out_type=jax.ShapeDtypeStruct((R, T, D), jnp.float32),
        mesh=plsc.VectorSubcoreMesh(core_axis_name="c", subcore_axis_name="s"),
        compiler_params=pltpu.CompilerParams(use_tc_tiling_on_sc=False),
        scratch_types=(
            [pltpu.VMEM((RPW, T), jnp.int32)]
            + [pltpu.VMEM((T, D), jnp.float32) for _ in range(2 * K)]
            + [pltpu.SemaphoreType.DMA] * 4
        ),
    )
    return call(token_ids.astype(jnp.int32), embedding_matrix)


# speedup vs baseline: 1.9121x; 1.3781x over previous
import jax
import jax.numpy as jnp
from jax import lax
from jax.experimental import pallas as pl
from jax.experimental.pallas import tpu as pltpu
from jax.experimental.pallas import tpu_sc as plsc

D = 32
T = 50
R = 16384
NC, NS = 2, 16
NW = NC * NS
RPW = R // NW
RT = RPW // 128
NSTEPS = T * RT
K = 4
NROUNDS = NSTEPS // K
BT = 129


def _emb_body(idx_hbm, table_hbm, out_hbm, idx_v, *scr):
    rows = [[scr[g * K + t] for t in range(K)] for g in range(2)]
    rowsT = [[scr[2 * K + g * K + t] for t in range(K)] for g in range(2)]
    gsem = [scr[4 * K], scr[4 * K + 1]]
    wsem = [scr[4 * K + 2], scr[4 * K + 3]]

    wid = lax.axis_index("s") * NC + lax.axis_index("c")
    base = wid * RPW
    pltpu.sync_copy(idx_hbm.at[:, pl.ds(base, RPW)], idx_v)

    lane = lax.iota(jnp.int32, 16)

    def g_copy(g, k, r):
        step = r * K + k
        t, rtl = step // RT, step % RT
        return pltpu.make_async_copy(
            table_hbm.at[idx_v.at[t, pl.ds(rtl * 128, 128)]],
            rows[g][k], gsem[g])

    def w_copies(g, k, r):
        step = r * K + k
        t, rtl = step // RT, step % RT
        rtg = wid * RT + rtl
        return [
            pltpu.make_async_copy(
                rowsT[g][k].at[pl.ds(dt * 8, 8), pl.ds(0, 128)],
                out_hbm.at[t, dt, rtg], wsem[g])
            for dt in range(4)
        ]

    def fire_g(g, r):
        for k in range(K):
            g_copy(g, k, r).start()

    def drain_g(g, r):
        for k in range(K):
            g_copy(g, k, r).wait()

    def fire_w(g, r):
        for k in range(K):
            for c in w_copies(g, k, r):
                c.start()

    def drain_w(g, r):
        for k in range(K):
            for c in w_copies(g, k, r):
                c.wait()

    def transpose(g, k):
        src, dst = rows[g][k], rowsT[g][k]

        def body(rl, carry):
            for d0 in (0, 16):
                v = src[rl, pl.ds(d0, 16)]
                plsc.store_scatter(
                    dst, [lane + d0, jnp.full((16,), rl, jnp.int32)], v)
            return carry

        lax.fori_loop(0, 128, body, 0)

    fire_g(0, 0)

    def process(g, r):
        @pl.when(r >= 1)
        def _():
            drain_w(1 - g, r - 1)

        @pl.when(r + 1 < NROUNDS)
        def _():
            fire_g(1 - g, r + 1)

        drain_g(g, r)
        for k in range(K):
            transpose(g, k)
        fire_w(g, r)

    def body(i, carry):
        process(0, 2 * i)
        process(1, 2 * i + 1)
        return carry

    lax.fori_loop(0, NROUNDS // 2, body, 0)
    drain_w((NROUNDS - 1) % 2, NROUNDS - 1)


def kernel(token_ids, embedding_matrix):
    call = pl.kernel(
        _emb_body,
        out_type=jax.ShapeDtypeStruct((T, 4, R // 128, 8, 128), jnp.float32),
        mesh=plsc.VectorSubcoreMesh(core_axis_name="c", subcore_axis_name="s"),
        compiler_params=pltpu.CompilerParams(
            use_tc_tiling_on_sc=False, needs_layout_passes=False),
        scratch_types=(
            [pltpu.VMEM((T, RPW), jnp.int32)]
            + [pltpu.VMEM((128, D), jnp.float32) for _ in range(2 * K)]
            + [pltpu.VMEM((D, BT), jnp.float32) for _ in range(2 * K)]
            + [pltpu.SemaphoreType.DMA] * 4
        ),
    )
    raw = call(token_ids.astype(jnp.int32).T, embedding_matrix)
    return raw.transpose(2, 4, 0, 1, 3).reshape(R, T, D)


# speedup vs baseline: 2.0168x; 1.0548x over previous
import jax
import jax.numpy as jnp
from jax import lax
from jax.experimental import pallas as pl
from jax.experimental.pallas import tpu as pltpu
from jax.experimental.pallas import tpu_sc as plsc

D = 32
T = 50
R = 16384
NC, NS = 2, 16
NW = NC * NS
RPW = R // NW
RT = RPW // 128
NSTEPS = T * RT
K = 4
NROUNDS = NSTEPS // K
BT = 129


def _emb_body(idx_hbm, table_hbm, out_hbm, idx_v, *scr):
    rows = [[scr[g * K + t] for t in range(K)] for g in range(2)]
    rowsT = [[scr[2 * K + g * K + t] for t in range(K)] for g in range(2)]
    gsem = [scr[4 * K], scr[4 * K + 1]]
    wsem = [scr[4 * K + 2], scr[4 * K + 3]]

    wid = lax.axis_index("s") * NC + lax.axis_index("c")
    base = wid * RPW
    pltpu.sync_copy(idx_hbm.at[:, pl.ds(base, RPW)], idx_v)

    lane = lax.iota(jnp.int32, 16)

    def g_copy(g, k, r):
        step = r * K + k
        t, rtl = step // RT, step % RT
        return pltpu.make_async_copy(
            table_hbm.at[idx_v.at[t, pl.ds(rtl * 128, 128)]],
            rows[g][k], gsem[g])

    def w_copies(g, k, r):
        step = r * K + k
        t, rtl = step // RT, step % RT
        rtg = wid * RT + rtl
        return [
            pltpu.make_async_copy(
                rowsT[g][k].at[pl.ds(dt * 8, 8), pl.ds(0, 128)],
                out_hbm.at[t, dt, rtg], wsem[g])
            for dt in range(4)
        ]

    def fire_g(g, r):
        for k in range(K):
            g_copy(g, k, r).start()

    def drain_w(g, r):
        for k in range(K):
            for c in w_copies(g, k, r):
                c.wait()

    def transpose(g, k):
        src, dst = rows[g][k], rowsT[g][k]

        def body(i, carry):
            for u in range(4):
                rl = i * 4 + u
                rsplat = jnp.full((16,), rl, jnp.int32)
                for d0 in (0, 16):
                    v = src[rl, pl.ds(d0, 16)]
                    plsc.store_scatter(dst, [lane + d0, rsplat], v)
            return carry

        lax.fori_loop(0, 32, body, 0)

    fire_g(0, 0)

    def process(g, r):
        @pl.when(r >= 1)
        def _():
            drain_w(1 - g, r - 1)

        @pl.when(r + 1 < NROUNDS)
        def _():
            fire_g(1 - g, r + 1)

        for k in range(K):
            g_copy(g, k, r).wait()
            transpose(g, k)
            for c in w_copies(g, k, r):
                c.start()

    def body(i, carry):
        process(0, 2 * i)
        process(1, 2 * i + 1)
        return carry

    lax.fori_loop(0, NROUNDS // 2, body, 0)
    drain_w((NROUNDS - 1) % 2, NROUNDS - 1)


def kernel(token_ids, embedding_matrix):
    call = pl.kernel(
        _emb_body,
        out_type=jax.ShapeDtypeStruct((T, 4, R // 128, 8, 128), jnp.float32),
        mesh=plsc.VectorSubcoreMesh(core_axis_name="c", subcore_axis_name="s"),
        compiler_params=pltpu.CompilerParams(
            use_tc_tiling_on_sc=False, needs_layout_passes=False),
        scratch_types=(
            [pltpu.VMEM((T, RPW), jnp.int32)]
            + [pltpu.VMEM((128, D), jnp.float32) for _ in range(2 * K)]
            + [pltpu.VMEM((D, BT), jnp.float32) for _ in range(2 * K)]
            + [pltpu.SemaphoreType.DMA] * 4
        ),
    )
    raw = call(token_ids.astype(jnp.int32).T, embedding_matrix)
    return raw.transpose(2, 4, 0, 1, 3).reshape(R, T, D)
